# Initial kernel scaffold; baseline (speedup 1.0000x reference)
import functools

import jax
import jax.numpy as jnp
from jax import lax
from jax.experimental import pallas as pl
from jax.experimental.pallas import tpu as pltpu

_N = 50000
_E = 800000
_Q = 100000
_NB = 128
_OC = 64
_NG = 64
_SET = 4

_SQRT1_2 = 0.7071067811865476


def _gelu(x):
    return 0.5 * x * (1.0 + lax.erf(x * _SQRT1_2))


def _dot(a, b):
    return jnp.dot(a, b, preferred_element_type=jnp.float32)


_BN1 = 500


def _emb_body(x_ref,
              win, bin_, whid, bhid, wproj, bproj, wout, bout,
              rin, rbin, rhid, rbhid, rproj, rbproj, rout, rbout,
              o_ref):
    xb = x_ref[...]
    h = _gelu(_dot(xb, win[...]) + bin_[...])
    h = _dot(h, whid[...]) + bhid[...]
    skip = _dot(xb, wproj[...]) + bproj[...]
    h = skip + _gelu(h)
    y = _dot(h, wout[...]) + bout[...]
    t = _gelu(y).reshape(_BN1, _SET, _NB)
    s = t.sum(axis=1)
    h2 = _gelu(_dot(s, rin[...]) + rbin[...])
    h2 = _dot(h2, rhid[...]) + rbhid[...]
    skip2 = _dot(s, rproj[...]) + rbproj[...]
    h2 = skip2 + _gelu(h2)
    o_ref[...] = _dot(h2, rout[...]) + rbout[...]


def _embedder(x2, wts):
    rb = _BN1 * _SET
    nblk = _N // _BN1
    full = lambda arr: pl.BlockSpec(arr.shape, lambda i: (0,) * arr.ndim)
    return pl.pallas_call(
        _emb_body,
        grid=(nblk,),
        in_specs=[pl.BlockSpec((rb, _NB), lambda i: (i, 0))] +
                 [full(w) for w in wts],
        out_specs=pl.BlockSpec((_BN1, _OC), lambda i: (i, 0)),
        out_shape=jax.ShapeDtypeStruct((_N, _OC), jnp.float32),
    )(x2, *wts)


_BN4 = 1000


def _conv_body(h_ref, s_ref, c_ref, wl, bl, wr, o_ref):
    h = h_ref[...]
    s = s_ref[...]
    cnt = c_ref[...]
    inv = 1.0 / jnp.maximum(cnt, 1.0)
    pre = _dot(s, wl[...]) * inv + bl[...] + _dot(h, wr[...])
    o_ref[...] = h + _gelu(pre)


def _conv_update(h, s, cnt_col, wl, bl, wr):
    nblk = _N // _BN4
    full = lambda arr: pl.BlockSpec(arr.shape, lambda i: (0,) * arr.ndim)
    return pl.pallas_call(
        _conv_body,
        grid=(nblk,),
        in_specs=[pl.BlockSpec((_BN4, _OC), lambda i: (i, 0)),
                  pl.BlockSpec((_BN4, _OC), lambda i: (i, 0)),
                  pl.BlockSpec((_BN4, 1), lambda i: (i, 0)),
                  full(wl), full(bl), full(wr)],
        out_specs=pl.BlockSpec((_BN4, _OC), lambda i: (i, 0)),
        out_shape=jax.ShapeDtypeStruct((_N, _OC), jnp.float32),
    )(h, s, cnt_col, wl, bl, wr)


_BN5 = 128
_NPAD = ((_N + _BN5 - 1) // _BN5) * _BN5
_NEG = -1e30


def _segmax_body(ids_ref, h_ref, o_ref):
    i = pl.program_id(0)

    @pl.when(i == 0)
    def _init():
        o_ref[...] = jnp.full((_NG, _OC), _NEG, jnp.float32)

    ids = ids_ref[...]
    iota = lax.broadcasted_iota(jnp.float32, (_NG, _BN5), 0)
    mask2 = iota == ids
    h = h_ref[...]
    masked = jnp.where(mask2[:, :, None], h[None, :, :], _NEG)
    o_ref[...] = jnp.maximum(o_ref[...], masked.max(axis=1))


def _segment_max(ids_f32_pad, h_pad):
    nblk = _NPAD // _BN5
    return pl.pallas_call(
        _segmax_body,
        grid=(nblk,),
        in_specs=[pl.BlockSpec((1, _BN5), lambda i: (i, 0)),
                  pl.BlockSpec((_BN5, _OC), lambda i: (i, 0))],
        out_specs=pl.BlockSpec((_NG, _OC), lambda i: (0, 0)),
        out_shape=jax.ShapeDtypeStruct((_NG, _OC), jnp.float32),
    )(ids_f32_pad, h_pad)


_BQ = 1000


def _head_body(se_ref, de_ref, g_ref,
               w1in, b1in, w1h, b1h, w1p, b1p, w1o, b1o,
               w2in, b2in, w2h, b2h, w2o,
               o_ref):
    se = se_ref[...]
    de = de_ref[...]
    g = g_ref[...]

    a_in = w1in[0:_OC, :]
    b_in = w1in[_OC:2 * _OC, :]
    g_in = w1in[2 * _OC:, :]
    a_p = w1p[0:_OC, :]
    b_p = w1p[_OC:2 * _OC, :]
    g_p = w1p[2 * _OC:, :]

    g_in_c = _dot(g, g_in) + b1in[...]
    g_p_c = _dot(g, g_p) + b1p[...]
    se_a = _dot(se, a_in)
    se_b = _dot(se, b_in)
    de_a = _dot(de, a_in)
    de_b = _dot(de, b_in)
    se_pa = _dot(se, a_p)
    se_pb = _dot(se, b_p)
    de_pa = _dot(de, a_p)
    de_pb = _dot(de, b_p)

    def mlp1(lin_in_val, proj_val):
        h = _gelu(lin_in_val)
        h = _dot(h, w1h[...]) + b1h[...]
        h = proj_val + _gelu(h)
        return _dot(h, w1o[...]) + b1o[...]

    def mlp2_pre(r):
        h = _gelu(_dot(r, w2in[...]) + b2in[...])
        h = _dot(h, w2h[...]) + b2h[...]
        return r + _gelu(h)

    r_xy = jax.nn.relu(mlp1(se_a + de_b + g_in_c, se_pa + de_pb + g_p_c))
    r_yx = jax.nn.relu(mlp1(de_a + se_b + g_in_c, de_pa + se_pb + g_p_c))
    diff = mlp2_pre(r_xy) - mlp2_pre(r_yx)
    o_ref[...] = _dot(diff, w2o[...])


def _query_head(se, de, g, wts):
    nblk = _Q // _BQ
    full = lambda arr: pl.BlockSpec(arr.shape, lambda i: (0,) * arr.ndim)
    return pl.pallas_call(
        _head_body,
        grid=(nblk,),
        in_specs=[pl.BlockSpec((_BQ, _OC), lambda i: (i, 0)),
                  pl.BlockSpec((_BQ, _OC), lambda i: (i, 0)),
                  pl.BlockSpec((_BQ, _OC), lambda i: (i, 0))] +
                 [full(w) for w in wts],
        out_specs=pl.BlockSpec((_BQ, 1), lambda i: (i, 0)),
        out_shape=jax.ShapeDtypeStruct((_Q, 1), jnp.float32),
    )(se, de, g, *wts)


def kernel(batch, x, edge_index, src_idx, dst_idx, params):
    p = params
    row = lambda b: b.reshape(1, -1)

    x2 = x.reshape(_N * _SET, _NB)
    pm = p['embedder']['mlp']
    pr = p['embedder']['rho']
    emb_wts = [pm['lin_in']['W'], row(pm['lin_in']['b']),
               pm['hid']['W'], row(pm['hid']['b']),
               pm['proj']['W'], row(pm['proj']['b']),
               pm['lin_out']['W'], row(pm['lin_out']['b']),
               pr['lin_in']['W'], row(pr['lin_in']['b']),
               pr['hid']['W'], row(pr['hid']['b']),
               pr['proj']['W'], row(pr['proj']['b']),
               pr['lin_out']['W'], row(pr['lin_out']['b'])]
    h = _embedder(x2, emb_wts)

    src = edge_index[0]
    dst = edge_index[1]
    cnt = jax.ops.segment_sum(jnp.ones((_E,), jnp.float32), dst,
                              num_segments=_N)
    cnt_col = cnt.reshape(_N, 1)
    for name in ('conv1', 'conv2', 'conv3'):
        cp = p[name]
        msg = jnp.take(h, src, axis=0)
        s = jax.ops.segment_sum(msg, dst, num_segments=_N)
        h = _conv_update(h, s, cnt_col, cp['Wl'], row(cp['bl']), cp['Wr'])

    ids_pad = jnp.concatenate(
        [batch, jnp.full((_NPAD - _N,), _NG, jnp.int32)]).astype(jnp.float32)
    h_pad = jnp.concatenate(
        [h, jnp.zeros((_NPAD - _N, _OC), jnp.float32)], axis=0)
    graph_emb = _segment_max(ids_pad.reshape(-1, _BN5), h_pad)

    se = jnp.take(h, src_idx, axis=0)
    de = jnp.take(h, dst_idx, axis=0)
    g = jnp.take(graph_emb, jnp.take(batch, src_idx), axis=0)

    p1 = p['lin1']
    p2 = p['lin2']
    head_wts = [p1['lin_in']['W'], row(p1['lin_in']['b']),
                p1['hid']['W'], row(p1['hid']['b']),
                p1['proj']['W'], row(p1['proj']['b']),
                p1['lin_out']['W'], row(p1['lin_out']['b']),
                p2['lin_in']['W'], row(p2['lin_in']['b']),
                p2['hid']['W'], row(p2['hid']['b']),
                p2['lin_out']['W']]
    return _query_head(se, de, g, head_wts)


# speedup vs baseline: 1.0501x; 1.0501x over previous
import functools

import jax
import jax.numpy as jnp
from jax import lax
from jax.experimental import pallas as pl
from jax.experimental.pallas import tpu as pltpu

_N = 50000
_E = 800000
_Q = 100000
_NB = 128
_OC = 64
_NG = 64
_SET = 4

_SQRT1_2 = 0.7071067811865476


def _erf(x):
    ax = jnp.abs(x)
    t = 1.0 / (1.0 + 0.3275911 * ax)
    poly = t * (0.254829592 + t * (-0.284496736 + t * (1.421413741 +
               t * (-1.453152027 + t * 1.061405429))))
    e = 1.0 - poly * jnp.exp(-ax * ax)
    return jnp.sign(x) * e


def _gelu(x):
    return 0.5 * x * (1.0 + _erf(x * _SQRT1_2))


def _dot(a, b):
    return jnp.dot(a, b, preferred_element_type=jnp.float32)


_BN1 = 400


def _emb_body(x_ref,
              win, bin_, whid, bhid, wproj, bproj, wout, bout,
              rin, rbin, rhid, rbhid, rproj, rbproj, rout, rbout,
              o_ref):
    xb = x_ref[...]
    h = _gelu(_dot(xb, win[...]) + bin_[...])
    h = _dot(h, whid[...]) + bhid[...]
    skip = _dot(xb, wproj[...]) + bproj[...]
    h = skip + _gelu(h)
    y = _dot(h, wout[...]) + bout[...]
    t = _gelu(y).reshape(_BN1, _SET, _NB)
    s = t.sum(axis=1)
    h2 = _gelu(_dot(s, rin[...]) + rbin[...])
    h2 = _dot(h2, rhid[...]) + rbhid[...]
    skip2 = _dot(s, rproj[...]) + rbproj[...]
    h2 = skip2 + _gelu(h2)
    o_ref[...] = _dot(h2, rout[...]) + rbout[...]


def _embedder(x2, wts):
    rb = _BN1 * _SET
    nblk = _N // _BN1
    full = lambda arr: pl.BlockSpec(arr.shape, lambda i: (0,) * arr.ndim)
    return pl.pallas_call(
        _emb_body,
        grid=(nblk,),
        in_specs=[pl.BlockSpec((rb, _NB), lambda i: (i, 0))] +
                 [full(w) for w in wts],
        out_specs=pl.BlockSpec((_BN1, _OC), lambda i: (i, 0)),
        out_shape=jax.ShapeDtypeStruct((_N, _OC), jnp.float32),
    )(x2, *wts)


_BN4 = 1000


def _conv_body(h_ref, s_ref, c_ref, wl, bl, wr, o_ref):
    h = h_ref[...]
    s = s_ref[...]
    cnt = c_ref[...]
    mean = s / jnp.maximum(cnt, 1.0)
    pre = _dot(mean, wl[...]) + bl[...] + _dot(h, wr[...])
    o_ref[...] = h + _gelu(pre)


def _conv_update(h, s, cnt_col, wl, bl, wr):
    nblk = _N // _BN4
    full = lambda arr: pl.BlockSpec(arr.shape, lambda i: (0,) * arr.ndim)
    return pl.pallas_call(
        _conv_body,
        grid=(nblk,),
        in_specs=[pl.BlockSpec((_BN4, _OC), lambda i: (i, 0)),
                  pl.BlockSpec((_BN4, _OC), lambda i: (i, 0)),
                  pl.BlockSpec((_BN4, 1), lambda i: (i, 0)),
                  full(wl), full(bl), full(wr)],
        out_specs=pl.BlockSpec((_BN4, _OC), lambda i: (i, 0)),
        out_shape=jax.ShapeDtypeStruct((_N, _OC), jnp.float32),
    )(h, s, cnt_col, wl, bl, wr)


_BN5 = 128
_NPAD = ((_N + _BN5 - 1) // _BN5) * _BN5
_NEG = -1e30


def _segmax_body(ids_ref, h_ref, o_ref):
    i = pl.program_id(0)

    @pl.when(i == 0)
    def _init():
        o_ref[...] = jnp.full((_NG, _OC), _NEG, jnp.float32)

    ids_col = ids_ref[...]
    h = h_ref[...]
    lo = ids_ref[0, 0]
    hi = ids_ref[_BN5 - 1, 0]
    for g in range(_NG):
        @pl.when((lo <= g) & (g <= hi))
        def _upd():
            m = ids_col == g
            mx = jnp.where(m, h, _NEG).max(axis=0)
            o_ref[g:g + 1, :] = jnp.maximum(o_ref[g:g + 1, :], mx[None, :])


def _segment_max(ids_col_pad, h_pad):
    nblk = _NPAD // _BN5
    return pl.pallas_call(
        _segmax_body,
        grid=(nblk,),
        in_specs=[pl.BlockSpec((_BN5, 1), lambda i: (i, 0)),
                  pl.BlockSpec((_BN5, _OC), lambda i: (i, 0))],
        out_specs=pl.BlockSpec((_NG, _OC), lambda i: (0, 0)),
        out_shape=jax.ShapeDtypeStruct((_NG, _OC), jnp.float32),
    )(ids_col_pad, h_pad)


_BQ = 1000


def _head_body(se_ref, de_ref, g_ref,
               w1in, b1in, w1h, b1h, w1p, b1p, w1o, b1o,
               w2in, b2in, w2h, b2h, w2o,
               o_ref):
    se = se_ref[...]
    de = de_ref[...]
    g = g_ref[...]

    a_in = w1in[0:_OC, :]
    b_in = w1in[_OC:2 * _OC, :]
    g_in = w1in[2 * _OC:, :]
    a_p = w1p[0:_OC, :]
    b_p = w1p[_OC:2 * _OC, :]
    g_p = w1p[2 * _OC:, :]

    g_in_c = _dot(g, g_in) + b1in[...]
    g_p_c = _dot(g, g_p) + b1p[...]
    se_a = _dot(se, a_in)
    se_b = _dot(se, b_in)
    de_a = _dot(de, a_in)
    de_b = _dot(de, b_in)
    se_pa = _dot(se, a_p)
    se_pb = _dot(se, b_p)
    de_pa = _dot(de, a_p)
    de_pb = _dot(de, b_p)

    def mlp1(lin_in_val, proj_val):
        h = _gelu(lin_in_val)
        h = _dot(h, w1h[...]) + b1h[...]
        h = proj_val + _gelu(h)
        return _dot(h, w1o[...]) + b1o[...]

    def mlp2(r):
        h = _gelu(_dot(r, w2in[...]) + b2in[...])
        h = _dot(h, w2h[...]) + b2h[...]
        t = r + _gelu(h)
        return _dot(t, w2o[...])

    r_xy = jax.nn.relu(mlp1(se_a + de_b + g_in_c, se_pa + de_pb + g_p_c))
    r_yx = jax.nn.relu(mlp1(de_a + se_b + g_in_c, de_pa + se_pb + g_p_c))
    o_ref[...] = mlp2(r_xy) - mlp2(r_yx)


def _query_head(se, de, g, wts):
    nblk = _Q // _BQ
    full = lambda arr: pl.BlockSpec(arr.shape, lambda i: (0,) * arr.ndim)
    return pl.pallas_call(
        _head_body,
        grid=(nblk,),
        in_specs=[pl.BlockSpec((_BQ, _OC), lambda i: (i, 0)),
                  pl.BlockSpec((_BQ, _OC), lambda i: (i, 0)),
                  pl.BlockSpec((_BQ, _OC), lambda i: (i, 0))] +
                 [full(w) for w in wts],
        out_specs=pl.BlockSpec((_BQ, 1), lambda i: (i, 0)),
        out_shape=jax.ShapeDtypeStruct((_Q, 1), jnp.float32),
    )(se, de, g, *wts)


def kernel(batch, x, edge_index, src_idx, dst_idx, params):
    p = params
    row = lambda b: b.reshape(1, -1)

    x2 = x.reshape(_N * _SET, _NB)
    pm = p['embedder']['mlp']
    pr = p['embedder']['rho']
    emb_wts = [pm['lin_in']['W'], row(pm['lin_in']['b']),
               pm['hid']['W'], row(pm['hid']['b']),
               pm['proj']['W'], row(pm['proj']['b']),
               pm['lin_out']['W'], row(pm['lin_out']['b']),
               pr['lin_in']['W'], row(pr['lin_in']['b']),
               pr['hid']['W'], row(pr['hid']['b']),
               pr['proj']['W'], row(pr['proj']['b']),
               pr['lin_out']['W'], row(pr['lin_out']['b'])]
    h = _embedder(x2, emb_wts)

    src = edge_index[0]
    dst = edge_index[1]
    cnt = jax.ops.segment_sum(jnp.ones((_E,), jnp.float32), dst,
                              num_segments=_N)
    cnt_col = cnt.reshape(_N, 1)
    for name in ('conv1', 'conv2', 'conv3'):
        cp = p[name]
        msg = jnp.take(h, src, axis=0)
        s = jax.ops.segment_sum(msg, dst, num_segments=_N)
        h = _conv_update(h, s, cnt_col, cp['Wl'], row(cp['bl']), cp['Wr'])

    ids_pad = jnp.concatenate(
        [batch, jnp.full((_NPAD - _N,), _NG, jnp.int32)])
    h_pad = jnp.concatenate(
        [h, jnp.zeros((_NPAD - _N, _OC), jnp.float32)], axis=0)
    graph_emb = _segment_max(ids_pad.reshape(-1, 1), h_pad)

    se = jnp.take(h, src_idx, axis=0)
    de = jnp.take(h, dst_idx, axis=0)
    g = jnp.take(graph_emb, jnp.take(batch, src_idx), axis=0)

    p1 = p['lin1']
    p2 = p['lin2']
    head_wts = [p1['lin_in']['W'], row(p1['lin_in']['b']),
                p1['hid']['W'], row(p1['hid']['b']),
                p1['proj']['W'], row(p1['proj']['b']),
                p1['lin_out']['W'], row(p1['lin_out']['b']),
                p2['lin_in']['W'], row(p2['lin_in']['b']),
                p2['hid']['W'], row(p2['hid']['b']),
                p2['lin_out']['W']]
    return _query_head(se, de, g, head_wts)


def _ref_lin(pp, x):
    y = x @ pp['W']
    if 'b' in pp:
        y = y + pp['b']
    return y


def _ref_mlp(pp, x):
    skip = x
    h = _ref_lin(pp['lin_in'], x)
    h = jax.nn.gelu(h, approximate=False)
    h = _ref_lin(pp['hid'], h)
    if 'proj' in pp:
        skip = _ref_lin(pp['proj'], skip)
    h = skip + jax.nn.gelu(h, approximate=False)
    return _ref_lin(pp['lin_out'], h)


_USE_PL = dict(emb=True, conv=True, segmax=True, head=True)


def kernel(batch, x, edge_index, src_idx, dst_idx, params):
    p = params
    row = lambda b: b.reshape(1, -1)
    if _USE_PL['emb']:
        x2 = x.reshape(_N * _SET, _NB)
        pm = p['embedder']['mlp']
        pr = p['embedder']['rho']
        emb_wts = [pm['lin_in']['W'], row(pm['lin_in']['b']),
                   pm['hid']['W'], row(pm['hid']['b']),
                   pm['proj']['W'], row(pm['proj']['b']),
                   pm['lin_out']['W'], row(pm['lin_out']['b']),
                   pr['lin_in']['W'], row(pr['lin_in']['b']),
                   pr['hid']['W'], row(pr['hid']['b']),
                   pr['proj']['W'], row(pr['proj']['b']),
                   pr['lin_out']['W'], row(pr['lin_out']['b'])]
        h = _embedder(x2, emb_wts)
    else:
        hh = jax.nn.gelu(_ref_mlp(p['embedder']['mlp'], x), approximate=False)
        h = _ref_mlp(p['embedder']['rho'], hh.sum(-2))
    src = edge_index[0]
    dst = edge_index[1]
    cnt = jax.ops.segment_sum(jnp.ones((_E,), jnp.float32), dst, num_segments=_N)
    cnt_col = cnt.reshape(_N, 1)
    for name in ('conv1', 'conv2', 'conv3'):
        cp = p[name]
        msg = jnp.take(h, src, axis=0)
        s = jax.ops.segment_sum(msg, dst, num_segments=_N)
        if _USE_PL['conv']:
            h = _conv_update(h, s, cnt_col, cp['Wl'], row(cp['bl']), cp['Wr'])
        else:
            mean = s / jnp.maximum(cnt, 1.0)[:, None]
            h = h + jax.nn.gelu(mean @ cp['Wl'] + cp['bl'] + h @ cp['Wr'],
                                approximate=False)
    if _USE_PL['segmax']:
        ids_pad = jnp.concatenate([batch, jnp.full((_NPAD - _N,), _NG, jnp.int32)])
        h_pad = jnp.concatenate([h, jnp.zeros((_NPAD - _N, _OC), jnp.float32)], axis=0)
        graph_emb = _segment_max(ids_pad.reshape(-1, 1), h_pad)
    else:
        graph_emb = jax.ops.segment_max(h, batch, num_segments=_NG)
    se = jnp.take(h, src_idx, axis=0)
    de = jnp.take(h, dst_idx, axis=0)
    g = jnp.take(graph_emb, jnp.take(batch, src_idx), axis=0)
    if _USE_PL['head']:
        p1 = p['lin1']
        p2 = p['lin2']
        head_wts = [p1['lin_in']['W'], row(p1['lin_in']['b']),
                    p1['hid']['W'], row(p1['hid']['b']),
                    p1['proj']['W'], row(p1['proj']['b']),
                    p1['lin_out']['W'], row(p1['lin_out']['b']),
                    p2['lin_in']['W'], row(p2['lin_in']['b']),
                    p2['hid']['W'], row(p2['hid']['b']),
                    p2['lin_out']['W']]
        out = _query_head(se, de, g, head_wts)
    else:
        xy = jnp.concatenate([se, de, g], axis=-1)
        yx = jnp.concatenate([de, se, g], axis=-1)

        def lin(z):
            return _ref_mlp(p['lin2'], jax.nn.relu(_ref_mlp(p['lin1'], z)))
        out = lin(xy) - lin(yx)

    def _copy_body(i_ref, o_ref):
        o_ref[...] = i_ref[...]
    out = pl.pallas_call(
        _copy_body,
        grid=(100,),
        in_specs=[pl.BlockSpec((_BQ, 1), lambda i: (i, 0))],
        out_specs=pl.BlockSpec((_BQ, 1), lambda i: (i, 0)),
        out_shape=jax.ShapeDtypeStruct(out.shape, out.dtype))(out)
    return out


# speedup vs baseline: 3.1994x; 3.0468x over previous
import functools

import jax
import jax.numpy as jnp
from jax import lax
from jax.experimental import pallas as pl
from jax.experimental.pallas import tpu as pltpu
from jax.experimental.pallas import tpu_sc as plsc

_N = 50000
_E = 800000
_Q = 100000
_NB = 128
_OC = 64
_NG = 64
_SET = 4

_SQRT1_2 = 0.7071067811865476


def _erf(x):
    ax = jnp.abs(x)
    t = 1.0 / (1.0 + 0.3275911 * ax)
    poly = t * (0.254829592 + t * (-0.284496736 + t * (1.421413741 +
               t * (-1.453152027 + t * 1.061405429))))
    e = 1.0 - poly * jnp.exp(-ax * ax)
    return jnp.sign(x) * e


def _gelu(x):
    return 0.5 * x * (1.0 + _erf(x * _SQRT1_2))


def _dot(a, b):
    return jnp.dot(a, b, preferred_element_type=jnp.float32)


_BN1 = 400


def _emb_body(x_ref,
              win, bin_, whid, bhid, wproj, bproj, wout, bout,
              rin, rbin, rhid, rbhid, rproj, rbproj, rout, rbout,
              o_ref):
    xb = x_ref[...]
    h = _gelu(_dot(xb, win[...]) + bin_[...])
    h = _dot(h, whid[...]) + bhid[...]
    skip = _dot(xb, wproj[...]) + bproj[...]
    h = skip + _gelu(h)
    y = _dot(h, wout[...]) + bout[...]
    t = _gelu(y).reshape(_BN1, _SET, _NB)
    s = t.sum(axis=1)
    h2 = _gelu(_dot(s, rin[...]) + rbin[...])
    h2 = _dot(h2, rhid[...]) + rbhid[...]
    skip2 = _dot(s, rproj[...]) + rbproj[...]
    h2 = skip2 + _gelu(h2)
    o_ref[...] = _dot(h2, rout[...]) + rbout[...]


def _embedder(x2, wts):
    rb = _BN1 * _SET
    nblk = _N // _BN1
    full = lambda arr: pl.BlockSpec(arr.shape, lambda i: (0,) * arr.ndim)
    return pl.pallas_call(
        _emb_body,
        grid=(nblk,),
        in_specs=[pl.BlockSpec((rb, _NB), lambda i: (i, 0))] +
                 [full(w) for w in wts],
        out_specs=pl.BlockSpec((_BN1, _OC), lambda i: (i, 0)),
        out_shape=jax.ShapeDtypeStruct((_N, _OC), jnp.float32),
    )(x2, *wts)


_CH = 128
_NCH = 392
_EPT = _NCH * _CH
_EPAD = 16 * _EPT
_HALF = 25088
_ACC = _HALF + 8
_TRASH = _HALF
_RPT = _HALF // 16

_SC_MESH = plsc.VectorSubcoreMesh(core_axis_name="c", subcore_axis_name="s")


def _agg_build(with_counts):
    out_type = [jax.ShapeDtypeStruct((2 * _HALF, _OC), jnp.float32)]
    if with_counts:
        out_type.append(jax.ShapeDtypeStruct((2 * _HALF,), jnp.float32))

    scratch = [
        pltpu.VMEM((_CH,), jnp.int32),
        pltpu.VMEM((_CH,), jnp.int32),
        pltpu.VMEM((_CH,), jnp.int32),
        pltpu.VMEM((_CH,), jnp.int32),
        pltpu.VMEM((_CH, _OC), jnp.float32),
        pltpu.VMEM((_CH, _OC), jnp.float32),
        pltpu.VMEM((_CH,), jnp.int32),
        pltpu.VMEM((_CH,), jnp.float32),
        pltpu.VMEM((_RPT,), jnp.float32),
        pltpu.VMEM_SHARED((_ACC, _OC), jnp.float32),
        pltpu.VMEM_SHARED((_ACC,), jnp.float32),
        pltpu.SemaphoreType.DMA,
        pltpu.SemaphoreType.DMA,
    ]

    def body(h_hbm, srcp, dstp, zer2, ones_hbm, *rest):
        if with_counts:
            out_hbm, cnt_hbm = rest[0], rest[1]
            rest = rest[2:]
        else:
            out_hbm = rest[0]
            rest = rest[1:]
        (sidx0, sidx1, didx0, didx1, msg0, msg1, dvec, ones_v, cbuf,
         accum, cacc, gsem0, gsem1) = rest

        c = lax.axis_index("c")
        s = lax.axis_index("s")
        lo = c * _HALF
        hi = jnp.minimum(lo + _HALF, _N)
        row0 = s * _RPT

        pltpu.sync_copy(zer2.at[pl.ds(row0, _RPT)],
                        accum.at[pl.ds(row0, _RPT)])
        if with_counts:
            pltpu.sync_copy(ones_hbm, ones_v)
            zv = jnp.zeros((16,), jnp.float32)

            def _zb(i, carry):
                cbuf[pl.ds(i * 16, 16)] = zv
                return carry
            lax.fori_loop(0, _RPT // 16, _zb, 0)
            pltpu.sync_copy(cbuf, cacc.at[pl.ds(row0, _RPT)])

        @pl.when(s == 0)
        def _zero_trash():
            pltpu.sync_copy(zer2.at[pl.ds(_HALF, 8)],
                            accum.at[pl.ds(_HALF, 8)])
            if with_counts:
                pltpu.sync_copy(cbuf.at[pl.ds(0, 8)],
                                cacc.at[pl.ds(_HALF, 8)])

        plsc.subcore_barrier()

        ebase = s * _EPT

        def fire(j, sidx, didx, msg, gsem):
            off = ebase + j * _CH
            pltpu.sync_copy(srcp.at[pl.ds(off, _CH)], sidx)
            pltpu.sync_copy(dstp.at[pl.ds(off, _CH)], dvec)
            for k in range(_CH // 16):
                d = dvec[pl.ds(k * 16, 16)]
                m = (d >= lo) & (d < hi)
                didx[pl.ds(k * 16, 16)] = jnp.where(m, d - lo, _TRASH)
            pltpu.async_copy(h_hbm.at[sidx], msg, gsem)

        def step(cur, sidxc, didxc, msgc, gsemc, sidxn, didxn, msgn, gsemn):
            nxt = cur + 1

            @pl.when(nxt < _NCH)
            def _prefetch():
                fire(nxt, sidxn, didxn, msgn, gsemn)

            pltpu.make_async_copy(h_hbm.at[sidxc], msgc, gsemc).wait()
            pltpu.sync_copy(msgc, accum.at[didxc], add=True)
            if with_counts:
                pltpu.sync_copy(ones_v, cacc.at[didxc], add=True)

        fire(0, sidx0, didx0, msg0, gsem0)

        def body_k(k, carry):
            cur = k * 2
            step(cur, sidx0, didx0, msg0, gsem0, sidx1, didx1, msg1, gsem1)
            step(cur + 1, sidx1, didx1, msg1, gsem1, sidx0, didx0, msg0,
                 gsem0)
            return carry

        lax.fori_loop(0, _NCH // 2, body_k, 0)

        plsc.subcore_barrier()

        pltpu.sync_copy(accum.at[pl.ds(row0, _RPT)],
                        out_hbm.at[pl.ds(c * _HALF + row0, _RPT)])
        if with_counts:
            pltpu.sync_copy(cacc.at[pl.ds(row0, _RPT)], cbuf)
            pltpu.sync_copy(cbuf, cnt_hbm.at[pl.ds(c * _HALF + row0, _RPT)])

    return pl.kernel(body, out_type=out_type, mesh=_SC_MESH,
                     scratch_types=scratch,
                     compiler_params=pltpu.CompilerParams(
                         use_tc_tiling_on_sc=False))


_agg_with_counts = _agg_build(True)
_agg_no_counts = _agg_build(False)


_BN4 = 1000


def _conv_body(h_ref, s_ref, c_ref, wl, bl, wr, o_ref):
    h = h_ref[...]
    s = s_ref[...]
    cnt = c_ref[...]
    mean = s / jnp.maximum(cnt, 1.0)
    pre = _dot(mean, wl[...]) + bl[...] + _dot(h, wr[...])
    o_ref[...] = h + _gelu(pre)


def _conv_update(h, s, cnt_col, wl, bl, wr):
    nblk = _N // _BN4
    full = lambda arr: pl.BlockSpec(arr.shape, lambda i: (0,) * arr.ndim)
    return pl.pallas_call(
        _conv_body,
        grid=(nblk,),
        in_specs=[pl.BlockSpec((_BN4, _OC), lambda i: (i, 0)),
                  pl.BlockSpec((_BN4, _OC), lambda i: (i, 0)),
                  pl.BlockSpec((_BN4, 1), lambda i: (i, 0)),
                  full(wl), full(bl), full(wr)],
        out_specs=pl.BlockSpec((_BN4, _OC), lambda i: (i, 0)),
        out_shape=jax.ShapeDtypeStruct((_N, _OC), jnp.float32),
    )(h, s, cnt_col, wl, bl, wr)


_BN5 = 128
_NPAD = ((_N + _BN5 - 1) // _BN5) * _BN5
_NEG = -1e30


def _segmax_body(ids_ref, h_ref, o_ref):
    i = pl.program_id(0)

    @pl.when(i == 0)
    def _init():
        o_ref[...] = jnp.full((_NG, _OC), _NEG, jnp.float32)

    ids_col = ids_ref[...]
    h = h_ref[...]
    lo = ids_ref[0, 0]
    hi = ids_ref[_BN5 - 1, 0]
    for g in range(_NG):
        @pl.when((lo <= g) & (g <= hi))
        def _upd():
            m = ids_col == g
            mx = jnp.where(m, h, _NEG).max(axis=0)
            o_ref[g:g + 1, :] = jnp.maximum(o_ref[g:g + 1, :], mx[None, :])


def _segment_max(ids_col_pad, h_pad):
    nblk = _NPAD // _BN5
    return pl.pallas_call(
        _segmax_body,
        grid=(nblk,),
        in_specs=[pl.BlockSpec((_BN5, 1), lambda i: (i, 0)),
                  pl.BlockSpec((_BN5, _OC), lambda i: (i, 0))],
        out_specs=pl.BlockSpec((_NG, _OC), lambda i: (0, 0)),
        out_shape=jax.ShapeDtypeStruct((_NG, _OC), jnp.float32),
    )(ids_col_pad, h_pad)


_BQ = 1000


def _head_body(se_ref, de_ref, g_ref,
               w1in, b1in, w1h, b1h, w1p, b1p, w1o, b1o,
               w2in, b2in, w2h, b2h, w2o,
               o_ref):
    se = se_ref[...]
    de = de_ref[...]
    g = g_ref[...]

    a_in = w1in[0:_OC, :]
    b_in = w1in[_OC:2 * _OC, :]
    g_in = w1in[2 * _OC:, :]
    a_p = w1p[0:_OC, :]
    b_p = w1p[_OC:2 * _OC, :]
    g_p = w1p[2 * _OC:, :]

    g_in_c = _dot(g, g_in) + b1in[...]
    g_p_c = _dot(g, g_p) + b1p[...]
    se_a = _dot(se, a_in)
    se_b = _dot(se, b_in)
    de_a = _dot(de, a_in)
    de_b = _dot(de, b_in)
    se_pa = _dot(se, a_p)
    se_pb = _dot(se, b_p)
    de_pa = _dot(de, a_p)
    de_pb = _dot(de, b_p)

    def mlp1(lin_in_val, proj_val):
        h = _gelu(lin_in_val)
        h = _dot(h, w1h[...]) + b1h[...]
        h = proj_val + _gelu(h)
        return _dot(h, w1o[...]) + b1o[...]

    def mlp2(r):
        h = _gelu(_dot(r, w2in[...]) + b2in[...])
        h = _dot(h, w2h[...]) + b2h[...]
        t = r + _gelu(h)
        return _dot(t, w2o[...])

    r_xy = jax.nn.relu(mlp1(se_a + de_b + g_in_c, se_pa + de_pb + g_p_c))
    r_yx = jax.nn.relu(mlp1(de_a + se_b + g_in_c, de_pa + se_pb + g_p_c))
    o_ref[...] = mlp2(r_xy) - mlp2(r_yx)


def _query_head(se, de, g, wts):
    nblk = _Q // _BQ
    full = lambda arr: pl.BlockSpec(arr.shape, lambda i: (0,) * arr.ndim)
    return pl.pallas_call(
        _head_body,
        grid=(nblk,),
        in_specs=[pl.BlockSpec((_BQ, _OC), lambda i: (i, 0)),
                  pl.BlockSpec((_BQ, _OC), lambda i: (i, 0)),
                  pl.BlockSpec((_BQ, _OC), lambda i: (i, 0))] +
                 [full(w) for w in wts],
        out_specs=pl.BlockSpec((_BQ, 1), lambda i: (i, 0)),
        out_shape=jax.ShapeDtypeStruct((_Q, 1), jnp.float32),
    )(se, de, g, *wts)


def kernel(batch, x, edge_index, src_idx, dst_idx, params):
    p = params
    row = lambda b: b.reshape(1, -1)

    x2 = x.reshape(_N * _SET, _NB)
    pm = p['embedder']['mlp']
    pr = p['embedder']['rho']
    emb_wts = [pm['lin_in']['W'], row(pm['lin_in']['b']),
               pm['hid']['W'], row(pm['hid']['b']),
               pm['proj']['W'], row(pm['proj']['b']),
               pm['lin_out']['W'], row(pm['lin_out']['b']),
               pr['lin_in']['W'], row(pr['lin_in']['b']),
               pr['hid']['W'], row(pr['hid']['b']),
               pr['proj']['W'], row(pr['proj']['b']),
               pr['lin_out']['W'], row(pr['lin_out']['b'])]
    h = _embedder(x2, emb_wts)

    src = edge_index[0]
    dst = edge_index[1]
    srcp = jnp.concatenate([src, jnp.zeros((_EPAD - _E,), jnp.int32)])
    dstp = jnp.concatenate([dst, jnp.full((_EPAD - _E,), -1, jnp.int32)])
    zer2 = jnp.zeros((_ACC, _OC), jnp.float32)
    ones128 = jnp.ones((_CH,), jnp.float32)

    cnt_col = None
    for li, name in enumerate(('conv1', 'conv2', 'conv3')):
        cp = p[name]
        if li == 0:
            s_pad, cnt_pad = _agg_with_counts(h, srcp, dstp, zer2, ones128)
            cnt_col = cnt_pad[:_N].reshape(_N, 1)
        else:
            res = _agg_no_counts(h, srcp, dstp, zer2, ones128)
            s_pad = res[0] if isinstance(res, (tuple, list)) else res
        s = s_pad[:_N]
        h = _conv_update(h, s, cnt_col, cp['Wl'], row(cp['bl']), cp['Wr'])

    ids_pad = jnp.concatenate(
        [batch, jnp.full((_NPAD - _N,), _NG, jnp.int32)])
    h_pad = jnp.concatenate(
        [h, jnp.zeros((_NPAD - _N, _OC), jnp.float32)], axis=0)
    graph_emb = _segment_max(ids_pad.reshape(-1, 1), h_pad)

    se = jnp.take(h, src_idx, axis=0)
    de = jnp.take(h, dst_idx, axis=0)
    g = jnp.take(graph_emb, jnp.take(batch, src_idx), axis=0)

    p1 = p['lin1']
    p2 = p['lin2']
    head_wts = [p1['lin_in']['W'], row(p1['lin_in']['b']),
                p1['hid']['W'], row(p1['hid']['b']),
                p1['proj']['W'], row(p1['proj']['b']),
                p1['lin_out']['W'], row(p1['lin_out']['b']),
                p2['lin_in']['W'], row(p2['lin_in']['b']),
                p2['hid']['W'], row(p2['hid']['b']),
                p2['lin_out']['W']]
    return _query_head(se, de, g, head_wts)


# speedup vs baseline: 3.3517x; 1.0476x over previous
import functools

import jax
import jax.numpy as jnp
from jax import lax
from jax.experimental import pallas as pl
from jax.experimental.pallas import tpu as pltpu
from jax.experimental.pallas import tpu_sc as plsc

_N = 50000
_E = 800000
_Q = 100000
_NB = 128
_OC = 64
_NG = 64
_SET = 4

_SQRT1_2 = 0.7071067811865476


def _erf(x):
    ax = jnp.abs(x)
    t = 1.0 / (1.0 + 0.3275911 * ax)
    poly = t * (0.254829592 + t * (-0.284496736 + t * (1.421413741 +
               t * (-1.453152027 + t * 1.061405429))))
    e = 1.0 - poly * jnp.exp(-ax * ax)
    return jnp.sign(x) * e


def _gelu(x):
    return 0.5 * x * (1.0 + _erf(x * _SQRT1_2))


def _dot(a, b):
    return jnp.dot(a, b, preferred_element_type=jnp.float32)


_BN1 = 400


def _emb_body(x_ref,
              win, bin_, whid, bhid, wproj, bproj, wout, bout,
              rin, rbin, rhid, rbhid, rproj, rbproj, rout, rbout,
              o_ref):
    xb = x_ref[...]
    h = _gelu(_dot(xb, win[...]) + bin_[...])
    h = _dot(h, whid[...]) + bhid[...]
    skip = _dot(xb, wproj[...]) + bproj[...]
    h = skip + _gelu(h)
    y = _dot(h, wout[...]) + bout[...]
    t = _gelu(y).reshape(_BN1, _SET, _NB)
    s = t.sum(axis=1)
    h2 = _gelu(_dot(s, rin[...]) + rbin[...])
    h2 = _dot(h2, rhid[...]) + rbhid[...]
    skip2 = _dot(s, rproj[...]) + rbproj[...]
    h2 = skip2 + _gelu(h2)
    o_ref[...] = _dot(h2, rout[...]) + rbout[...]


def _embedder(x2, wts):
    rb = _BN1 * _SET
    nblk = _N // _BN1
    full = lambda arr: pl.BlockSpec(arr.shape, lambda i: (0,) * arr.ndim)
    return pl.pallas_call(
        _emb_body,
        grid=(nblk,),
        in_specs=[pl.BlockSpec((rb, _NB), lambda i: (i, 0))] +
                 [full(w) for w in wts],
        out_specs=pl.BlockSpec((_BN1, _OC), lambda i: (i, 0)),
        out_shape=jax.ShapeDtypeStruct((_N, _OC), jnp.float32),
    )(x2, *wts)


_CH = 128
_NCH = 392
_EPT = _NCH * _CH
_EPAD = 16 * _EPT
_HALF = 25088
_ACC = _HALF + 8
_TRASH = _HALF
_RPT = _HALF // 16

_SC_MESH = plsc.VectorSubcoreMesh(core_axis_name="c", subcore_axis_name="s")


def _agg_build(with_counts):
    out_type = [jax.ShapeDtypeStruct((2 * _HALF, _OC), jnp.float32)]
    if with_counts:
        out_type.append(jax.ShapeDtypeStruct((2 * _HALF,), jnp.float32))

    scratch = [
        pltpu.VMEM((_CH,), jnp.int32),
        pltpu.VMEM((_CH,), jnp.int32),
        pltpu.VMEM((_CH,), jnp.int32),
        pltpu.VMEM((_CH,), jnp.int32),
        pltpu.VMEM((_CH, _OC), jnp.float32),
        pltpu.VMEM((_CH, _OC), jnp.float32),
        pltpu.VMEM((_CH,), jnp.int32),
        pltpu.VMEM((_CH,), jnp.float32),
        pltpu.VMEM((_RPT,), jnp.float32),
        pltpu.VMEM_SHARED((_ACC, _OC), jnp.float32),
        pltpu.VMEM_SHARED((_ACC,), jnp.float32),
        pltpu.SemaphoreType.DMA,
        pltpu.SemaphoreType.DMA,
    ]

    def body(h_hbm, srcp, dstp, zer2, ones_hbm, *rest):
        if with_counts:
            out_hbm, cnt_hbm = rest[0], rest[1]
            rest = rest[2:]
        else:
            out_hbm = rest[0]
            rest = rest[1:]
        (sidx0, sidx1, didx0, didx1, msg0, msg1, dvec, ones_v, cbuf,
         accum, cacc, gsem0, gsem1) = rest

        c = lax.axis_index("c")
        s = lax.axis_index("s")
        lo = c * _HALF
        hi = jnp.minimum(lo + _HALF, _N)
        row0 = s * _RPT

        pltpu.sync_copy(zer2.at[pl.ds(row0, _RPT)],
                        accum.at[pl.ds(row0, _RPT)])
        if with_counts:
            pltpu.sync_copy(ones_hbm, ones_v)
            zv = jnp.zeros((16,), jnp.float32)

            def _zb(i, carry):
                cbuf[pl.ds(i * 16, 16)] = zv
                return carry
            lax.fori_loop(0, _RPT // 16, _zb, 0)
            pltpu.sync_copy(cbuf, cacc.at[pl.ds(row0, _RPT)])

        @pl.when(s == 0)
        def _zero_trash():
            pltpu.sync_copy(zer2.at[pl.ds(_HALF, 8)],
                            accum.at[pl.ds(_HALF, 8)])
            if with_counts:
                pltpu.sync_copy(cbuf.at[pl.ds(0, 8)],
                                cacc.at[pl.ds(_HALF, 8)])

        plsc.subcore_barrier()

        ebase = s * _EPT

        def fire(j, sidx, didx, msg, gsem):
            off = ebase + j * _CH
            pltpu.sync_copy(srcp.at[pl.ds(off, _CH)], sidx)
            pltpu.sync_copy(dstp.at[pl.ds(off, _CH)], dvec)
            for k in range(_CH // 16):
                d = dvec[pl.ds(k * 16, 16)]
                m = (d >= lo) & (d < hi)
                didx[pl.ds(k * 16, 16)] = jnp.where(m, d - lo, _TRASH)
            pltpu.async_copy(h_hbm.at[sidx], msg, gsem)

        def step(cur, sidxc, didxc, msgc, gsemc, sidxn, didxn, msgn, gsemn):
            nxt = cur + 1

            @pl.when(nxt < _NCH)
            def _prefetch():
                fire(nxt, sidxn, didxn, msgn, gsemn)

            pltpu.make_async_copy(h_hbm.at[sidxc], msgc, gsemc).wait()
            pltpu.sync_copy(msgc, accum.at[didxc], add=True)
            if with_counts:
                pltpu.sync_copy(ones_v, cacc.at[didxc], add=True)

        fire(0, sidx0, didx0, msg0, gsem0)

        def body_k(k, carry):
            cur = k * 2
            step(cur, sidx0, didx0, msg0, gsem0, sidx1, didx1, msg1, gsem1)
            step(cur + 1, sidx1, didx1, msg1, gsem1, sidx0, didx0, msg0,
                 gsem0)
            return carry

        lax.fori_loop(0, _NCH // 2, body_k, 0)

        plsc.subcore_barrier()

        pltpu.sync_copy(accum.at[pl.ds(row0, _RPT)],
                        out_hbm.at[pl.ds(c * _HALF + row0, _RPT)])
        if with_counts:
            pltpu.sync_copy(cacc.at[pl.ds(row0, _RPT)], cbuf)
            pltpu.sync_copy(cbuf, cnt_hbm.at[pl.ds(c * _HALF + row0, _RPT)])

    return pl.kernel(body, out_type=out_type, mesh=_SC_MESH,
                     scratch_types=scratch,
                     compiler_params=pltpu.CompilerParams(
                         use_tc_tiling_on_sc=False))


_agg_with_counts = _agg_build(True)
_agg_no_counts = _agg_build(False)


_QPAD = 102400
_QPT = _QPAD // 32
_QNCH = _QPT // _CH


def _qgather_body(h_hbm, ge_hbm, batch_hbm, si_hbm, di_hbm,
                  se_hbm, de_hbm, g_hbm,
                  siv, div_, bvec, seb, deb, gb, sem, sem2):
    c = lax.axis_index("c")
    s = lax.axis_index("s")
    w = s * 2 + c
    base = w * _QPT

    def step(j, carry):
        off = base + j * _CH
        pltpu.sync_copy(si_hbm.at[pl.ds(off, _CH)], siv)
        pltpu.sync_copy(di_hbm.at[pl.ds(off, _CH)], div_)
        pltpu.async_copy(batch_hbm.at[siv], bvec, sem2).wait()
        pltpu.async_copy(h_hbm.at[siv], seb, sem)
        pltpu.async_copy(h_hbm.at[div_], deb, sem)
        pltpu.async_copy(ge_hbm.at[bvec], gb, sem)
        pltpu.make_async_copy(h_hbm.at[siv], seb, sem).wait()
        pltpu.make_async_copy(h_hbm.at[div_], deb, sem).wait()
        pltpu.make_async_copy(ge_hbm.at[bvec], gb, sem).wait()
        pltpu.sync_copy(seb, se_hbm.at[pl.ds(off, _CH)])
        pltpu.sync_copy(deb, de_hbm.at[pl.ds(off, _CH)])
        pltpu.sync_copy(gb, g_hbm.at[pl.ds(off, _CH)])
        return carry

    lax.fori_loop(0, _QNCH, step, 0)


_qgather = pl.kernel(
    _qgather_body,
    out_type=[jax.ShapeDtypeStruct((_QPAD, _OC), jnp.float32)] * 3,
    mesh=_SC_MESH,
    scratch_types=[
        pltpu.VMEM((_CH,), jnp.int32),
        pltpu.VMEM((_CH,), jnp.int32),
        pltpu.VMEM((_CH,), jnp.int32),
        pltpu.VMEM((_CH, _OC), jnp.float32),
        pltpu.VMEM((_CH, _OC), jnp.float32),
        pltpu.VMEM((_CH, _OC), jnp.float32),
        pltpu.SemaphoreType.DMA,
        pltpu.SemaphoreType.DMA,
    ],
    compiler_params=pltpu.CompilerParams(use_tc_tiling_on_sc=False))


_BN4 = 1000


def _conv_body(h_ref, s_ref, c_ref, wl, bl, wr, o_ref):
    h = h_ref[...]
    s = s_ref[...]
    cnt = c_ref[...]
    mean = s / jnp.maximum(cnt, 1.0)
    pre = _dot(mean, wl[...]) + bl[...] + _dot(h, wr[...])
    o_ref[...] = h + _gelu(pre)


def _conv_update(h, s, cnt_col, wl, bl, wr):
    nblk = _N // _BN4
    full = lambda arr: pl.BlockSpec(arr.shape, lambda i: (0,) * arr.ndim)
    return pl.pallas_call(
        _conv_body,
        grid=(nblk,),
        in_specs=[pl.BlockSpec((_BN4, _OC), lambda i: (i, 0)),
                  pl.BlockSpec((_BN4, _OC), lambda i: (i, 0)),
                  pl.BlockSpec((_BN4, 1), lambda i: (i, 0)),
                  full(wl), full(bl), full(wr)],
        out_specs=pl.BlockSpec((_BN4, _OC), lambda i: (i, 0)),
        out_shape=jax.ShapeDtypeStruct((_N, _OC), jnp.float32),
    )(h, s, cnt_col, wl, bl, wr)


_BN5 = 128
_NPAD = ((_N + _BN5 - 1) // _BN5) * _BN5
_NEG = -1e30


def _segmax_body(ids_ref, h_ref, o_ref):
    i = pl.program_id(0)

    @pl.when(i == 0)
    def _init():
        o_ref[...] = jnp.full((_NG, _OC), _NEG, jnp.float32)

    ids_col = ids_ref[...]
    h = h_ref[...]
    lo = ids_ref[0, 0]
    hi = ids_ref[_BN5 - 1, 0]
    for g in range(_NG):
        @pl.when((lo <= g) & (g <= hi))
        def _upd():
            m = ids_col == g
            mx = jnp.where(m, h, _NEG).max(axis=0)
            o_ref[g:g + 1, :] = jnp.maximum(o_ref[g:g + 1, :], mx[None, :])


def _segment_max(ids_col_pad, h_pad):
    nblk = _NPAD // _BN5
    return pl.pallas_call(
        _segmax_body,
        grid=(nblk,),
        in_specs=[pl.BlockSpec((_BN5, 1), lambda i: (i, 0)),
                  pl.BlockSpec((_BN5, _OC), lambda i: (i, 0))],
        out_specs=pl.BlockSpec((_NG, _OC), lambda i: (0, 0)),
        out_shape=jax.ShapeDtypeStruct((_NG, _OC), jnp.float32),
    )(ids_col_pad, h_pad)


_BQ = 1024


def _head_body(se_ref, de_ref, g_ref,
               w1in, b1in, w1h, b1h, w1p, b1p, w1o, b1o,
               w2in, b2in, w2h, b2h, w2o,
               o_ref):
    se = se_ref[...]
    de = de_ref[...]
    g = g_ref[...]

    a_in = w1in[0:_OC, :]
    b_in = w1in[_OC:2 * _OC, :]
    g_in = w1in[2 * _OC:, :]
    a_p = w1p[0:_OC, :]
    b_p = w1p[_OC:2 * _OC, :]
    g_p = w1p[2 * _OC:, :]

    g_in_c = _dot(g, g_in) + b1in[...]
    g_p_c = _dot(g, g_p) + b1p[...]
    se_a = _dot(se, a_in)
    se_b = _dot(se, b_in)
    de_a = _dot(de, a_in)
    de_b = _dot(de, b_in)
    se_pa = _dot(se, a_p)
    se_pb = _dot(se, b_p)
    de_pa = _dot(de, a_p)
    de_pb = _dot(de, b_p)

    def mlp1(lin_in_val, proj_val):
        h = _gelu(lin_in_val)
        h = _dot(h, w1h[...]) + b1h[...]
        h = proj_val + _gelu(h)
        return _dot(h, w1o[...]) + b1o[...]

    def mlp2(r):
        h = _gelu(_dot(r, w2in[...]) + b2in[...])
        h = _dot(h, w2h[...]) + b2h[...]
        t = r + _gelu(h)
        return _dot(t, w2o[...])

    r_xy = jax.nn.relu(mlp1(se_a + de_b + g_in_c, se_pa + de_pb + g_p_c))
    r_yx = jax.nn.relu(mlp1(de_a + se_b + g_in_c, de_pa + se_pb + g_p_c))
    o_ref[...] = mlp2(r_xy) - mlp2(r_yx)


def _query_head(se, de, g, wts):
    nblk = _QPAD // _BQ
    full = lambda arr: pl.BlockSpec(arr.shape, lambda i: (0,) * arr.ndim)
    return pl.pallas_call(
        _head_body,
        grid=(nblk,),
        in_specs=[pl.BlockSpec((_BQ, _OC), lambda i: (i, 0)),
                  pl.BlockSpec((_BQ, _OC), lambda i: (i, 0)),
                  pl.BlockSpec((_BQ, _OC), lambda i: (i, 0))] +
                 [full(w) for w in wts],
        out_specs=pl.BlockSpec((_BQ, 1), lambda i: (i, 0)),
        out_shape=jax.ShapeDtypeStruct((_QPAD, 1), jnp.float32),
    )(se, de, g, *wts)


def kernel(batch, x, edge_index, src_idx, dst_idx, params):
    p = params
    row = lambda b: b.reshape(1, -1)

    x2 = x.reshape(_N * _SET, _NB)
    pm = p['embedder']['mlp']
    pr = p['embedder']['rho']
    emb_wts = [pm['lin_in']['W'], row(pm['lin_in']['b']),
               pm['hid']['W'], row(pm['hid']['b']),
               pm['proj']['W'], row(pm['proj']['b']),
               pm['lin_out']['W'], row(pm['lin_out']['b']),
               pr['lin_in']['W'], row(pr['lin_in']['b']),
               pr['hid']['W'], row(pr['hid']['b']),
               pr['proj']['W'], row(pr['proj']['b']),
               pr['lin_out']['W'], row(pr['lin_out']['b'])]
    h = _embedder(x2, emb_wts)

    src = edge_index[0]
    dst = edge_index[1]
    srcp = jnp.concatenate([src, jnp.zeros((_EPAD - _E,), jnp.int32)])
    dstp = jnp.concatenate([dst, jnp.full((_EPAD - _E,), -1, jnp.int32)])
    zer2 = jnp.zeros((_ACC, _OC), jnp.float32)
    ones128 = jnp.ones((_CH,), jnp.float32)

    cnt_col = None
    for li, name in enumerate(('conv1', 'conv2', 'conv3')):
        cp = p[name]
        if li == 0:
            s_pad, cnt_pad = _agg_with_counts(h, srcp, dstp, zer2, ones128)
            cnt_col = cnt_pad[:_N].reshape(_N, 1)
        else:
            res = _agg_no_counts(h, srcp, dstp, zer2, ones128)
            s_pad = res[0] if isinstance(res, (tuple, list)) else res
        s = s_pad[:_N]
        h = _conv_update(h, s, cnt_col, cp['Wl'], row(cp['bl']), cp['Wr'])

    ids_pad = jnp.concatenate(
        [batch, jnp.full((_NPAD - _N,), _NG, jnp.int32)])
    h_pad = jnp.concatenate(
        [h, jnp.zeros((_NPAD - _N, _OC), jnp.float32)], axis=0)
    graph_emb = _segment_max(ids_pad.reshape(-1, 1), h_pad)

    qpad = jnp.zeros((_QPAD - _Q,), jnp.int32)
    sip = jnp.concatenate([src_idx, qpad])
    dip = jnp.concatenate([dst_idx, qpad])
    se, de, g = _qgather(h, graph_emb, batch, sip, dip)

    p1 = p['lin1']
    p2 = p['lin2']
    head_wts = [p1['lin_in']['W'], row(p1['lin_in']['b']),
                p1['hid']['W'], row(p1['hid']['b']),
                p1['proj']['W'], row(p1['proj']['b']),
                p1['lin_out']['W'], row(p1['lin_out']['b']),
                p2['lin_in']['W'], row(p2['lin_in']['b']),
                p2['hid']['W'], row(p2['hid']['b']),
                p2['lin_out']['W']]
    return _query_head(se, de, g, head_wts)[:_Q]
